# baseline (speedup 1.0000x reference)
import jax
import jax.numpy as jnp
from jax.experimental import pallas as pl
from jax.experimental.pallas import tpu as pltpu

_CB = 2048
_N_BUF = 3


def _make_body(n_steps, n_rows):
    n_full = n_steps - 1
    tail = n_rows - n_full * _CB

    def body(x_ref, f_ref, o_hbm, xn_scr, obuf, sems):
        j = pl.program_id(0)
        slot = jax.lax.rem(j, _N_BUF)

        @pl.when(j == 0)
        def _():
            x = x_ref[...]
            nrm = jnp.sqrt(jnp.sum(x * x, axis=1, keepdims=True))
            xn_scr[...] = (x / jnp.maximum(nrm, 1e-12)).astype(jnp.bfloat16)

        def copy(step, s, rows):
            return pltpu.make_async_copy(
                obuf.at[s, pl.ds(0, rows)],
                o_hbm.at[pl.ds(step * _CB, rows), :],
                sems.at[s],
            )

        @pl.when(j >= _N_BUF)
        def _():
            copy(j - _N_BUF, slot, _CB).wait()

        obuf[slot] = jax.lax.dot_general(
            f_ref[...].astype(jnp.bfloat16),
            xn_scr[...],
            (((1,), (1,)), ((), ())),
            preferred_element_type=jnp.float32,
        )

        @pl.when(j < n_full)
        def _():
            copy(j, slot, _CB).start()

        @pl.when(j == n_steps - 1)
        def _():
            copy(j, slot, tail).start()
            for step in range(max(0, n_steps - _N_BUF), n_steps - 1):
                copy(step, step % _N_BUF, _CB).wait()
            copy(n_steps - 1, (n_steps - 1) % _N_BUF, tail).wait()

    return body


def kernel(bn_global_x, targets, features):
    b, f = bn_global_x.shape
    n = features.shape[0]
    n_steps = pl.cdiv(n, _CB)
    out_t = pl.pallas_call(
        _make_body(n_steps, n),
        grid=(n_steps,),
        in_specs=[
            pl.BlockSpec((b, f), lambda j: (0, 0)),
            pl.BlockSpec((_CB, f), lambda j: (j, 0)),
        ],
        out_specs=pl.BlockSpec(memory_space=pl.ANY),
        out_shape=jax.ShapeDtypeStruct((n, b), jnp.float32),
        scratch_shapes=[
            pltpu.VMEM((b, f), jnp.bfloat16),
            pltpu.VMEM((_N_BUF, _CB, b), jnp.float32),
            pltpu.SemaphoreType.DMA((_N_BUF,)),
        ],
        compiler_params=pltpu.CompilerParams(dimension_semantics=("arbitrary",)),
    )(bn_global_x, features)
    return (out_t.T, features)


# speedup vs baseline: 1.0165x; 1.0165x over previous
import jax
import jax.numpy as jnp
from jax.experimental import pallas as pl
from jax.experimental.pallas import tpu as pltpu

_CB = 6144
_N_BUF = 2


def _make_body(n_steps, n_rows):
    n_full = n_steps - 1
    tail = n_rows - n_full * _CB

    def body(x_ref, f_ref, o_hbm, xn_scr, obuf, sems):
        j = pl.program_id(0)
        slot = jax.lax.rem(j, _N_BUF)

        @pl.when(j == 0)
        def _():
            x = x_ref[...]
            nrm = jnp.sqrt(jnp.sum(x * x, axis=1, keepdims=True))
            xn_scr[...] = (x / jnp.maximum(nrm, 1e-12)).astype(jnp.bfloat16)

        def copy(step, s, rows):
            return pltpu.make_async_copy(
                obuf.at[s, pl.ds(0, rows)],
                o_hbm.at[pl.ds(step * _CB, rows), :],
                sems.at[s],
            )

        @pl.when(j >= _N_BUF)
        def _():
            copy(j - _N_BUF, slot, _CB).wait()

        obuf[slot] = jax.lax.dot_general(
            f_ref[...].astype(jnp.bfloat16),
            xn_scr[...],
            (((1,), (1,)), ((), ())),
            preferred_element_type=jnp.float32,
        )

        @pl.when(j < n_full)
        def _():
            copy(j, slot, _CB).start()

        @pl.when(j == n_steps - 1)
        def _():
            copy(j, slot, tail).start()
            for step in range(max(0, n_steps - _N_BUF), n_steps - 1):
                copy(step, step % _N_BUF, _CB).wait()
            copy(n_steps - 1, (n_steps - 1) % _N_BUF, tail).wait()

    return body


def kernel(bn_global_x, targets, features):
    b, f = bn_global_x.shape
    n = features.shape[0]
    n_steps = pl.cdiv(n, _CB)
    out_t = pl.pallas_call(
        _make_body(n_steps, n),
        grid=(n_steps,),
        in_specs=[
            pl.BlockSpec((b, f), lambda j: (0, 0)),
            pl.BlockSpec((_CB, f), lambda j: (j, 0)),
        ],
        out_specs=pl.BlockSpec(memory_space=pl.ANY),
        out_shape=jax.ShapeDtypeStruct((n, b), jnp.float32),
        scratch_shapes=[
            pltpu.VMEM((b, f), jnp.bfloat16),
            pltpu.VMEM((_N_BUF, _CB, b), jnp.float32),
            pltpu.SemaphoreType.DMA((_N_BUF,)),
        ],
        compiler_params=pltpu.CompilerParams(dimension_semantics=("arbitrary",)),
    )(bn_global_x, features)
    return (out_t.T, features)
